# baseline (speedup 1.0000x reference)
import jax
import jax.numpy as jnp
from jax.experimental import pallas as pl
from jax.experimental.pallas import tpu as pltpu

N = 20000
C = 10
BD = 9
PRE = 1024
THR = 0.5
MAXN = 500
STH = 0.05
NPAD = 20480
R = 160
L = 128


def _kern(scores_ref, boxes_ref,
          obox_ref, oscore_ref, olab_ref,
          m_ref, lab_ref, selbox_ref, rm_ref,
          x1T, x2T, y1T, y2T, areaT, scT, labT, keepT):
    s0 = jax.nn.sigmoid(scores_ref[0])
    m = jnp.where(s0 > STH, s0, 0.0)
    lab = jnp.zeros((R, L), jnp.int32)
    for c in range(1, C):
        sc = jax.nn.sigmoid(scores_ref[c])
        th = jnp.where(sc > STH, sc, 0.0)
        lab = jnp.where(th > m, c, lab)
        m = jnp.maximum(m, th)
    flat = (jax.lax.broadcasted_iota(jnp.int32, (R, L), 0) * L
            + jax.lax.broadcasted_iota(jnp.int32, (R, L), 1))
    m = jnp.where(flat < N, m, -1.0)
    m_ref[...] = m
    lab_ref[...] = lab
    rm_ref[...] = jnp.max(m, axis=1, keepdims=True)
    riota = jax.lax.broadcasted_iota(jnp.int32, (R, 1), 0)
    lane128 = jax.lax.broadcasted_iota(jnp.int32, (1, L), 1)

    lane = jax.lax.broadcasted_iota(jnp.int32, (1, PRE), 1)
    zeros = jnp.zeros((1, PRE), jnp.float32)
    x1T[...] = zeros
    x2T[...] = zeros
    y1T[...] = zeros
    y2T[...] = zeros
    areaT[...] = zeros
    scT[...] = zeros
    labT[...] = jnp.zeros((1, PRE), jnp.int32)
    keepT[...] = zeros + 1.0

    def sel_body(k, carry):
        rm = rm_ref[...]
        v = jnp.max(rm)
        r = jnp.min(jnp.where(rm == v, riota, R))
        srow = m_ref[pl.ds(r, 1), :]
        cc = jnp.min(jnp.where(srow == v, lane128, L))
        idx = r * L + cc
        row = boxes_ref[pl.ds(idx, 1), :]
        selbox_ref[pl.ds(k, 1), :] = row
        x = row[:, 0:1]
        y = row[:, 1:2]
        w = jnp.abs(row[:, 3:4]) + 1e-3
        ln = jnp.abs(row[:, 4:5]) + 1e-3
        msk = lane == k
        x1T[...] = jnp.where(msk, x - w / 2.0, x1T[...])
        x2T[...] = jnp.where(msk, x + w / 2.0, x2T[...])
        y1T[...] = jnp.where(msk, y - ln / 2.0, y1T[...])
        y2T[...] = jnp.where(msk, y + ln / 2.0, y2T[...])
        areaT[...] = jnp.where(msk, w * ln, areaT[...])
        scT[...] = jnp.where(msk, v, scT[...])
        lab_row = lab_ref[pl.ds(r, 1), :]
        lv = jnp.sum(jnp.where(lane128 == cc, lab_row, 0))
        labT[...] = jnp.where(msk, lv, labT[...])
        newrow = jnp.where(lane128 == cc, -1.0, srow)
        m_ref[pl.ds(r, 1), :] = newrow
        rm_ref[pl.ds(r, 1), :] = jnp.max(newrow, axis=1, keepdims=True)
        return carry

    jax.lax.fori_loop(0, PRE, sel_body, 0)

    def nms_body(i, carry):
        kt = keepT[...]
        onehot = lane == i
        ki = jnp.sum(jnp.where(onehot, kt, 0.0))
        x1i = jnp.sum(jnp.where(onehot, x1T[...], 0.0))
        x2i = jnp.sum(jnp.where(onehot, x2T[...], 0.0))
        y1i = jnp.sum(jnp.where(onehot, y1T[...], 0.0))
        y2i = jnp.sum(jnp.where(onehot, y2T[...], 0.0))
        ai = jnp.sum(jnp.where(onehot, areaT[...], 0.0))
        ix = jnp.maximum(0.0, jnp.minimum(x2i, x2T[...]) - jnp.maximum(x1i, x1T[...]))
        iy = jnp.maximum(0.0, jnp.minimum(y2i, y2T[...]) - jnp.maximum(y1i, y1T[...]))
        inter = ix * iy
        union = ai + areaT[...] - inter
        iou = inter / jnp.maximum(union, 1e-6)
        supp = (iou > THR) & (lane > i) & (ki > 0.0)
        keepT[...] = jnp.where(supp, 0.0, kt)
        return carry

    jax.lax.fori_loop(0, PRE, nms_body, 0)

    scT[...] = jnp.where(keepT[...] > 0.0, scT[...], 0.0)

    def out_body(k, carry):
        f = scT[...]
        v = jnp.max(f)
        idx = jnp.min(jnp.where(f == v, lane, PRE))
        obox_ref[pl.ds(k, 1), :] = selbox_ref[pl.ds(idx, 1), :]
        oscore_ref[pl.ds(k, 1), :] = jnp.full((1, 1), v, jnp.float32)
        lv = jnp.sum(jnp.where(lane == idx, labT[...], 0))
        olab_ref[pl.ds(k, 1), :] = jnp.full((1, 1), lv, jnp.int32)
        scT[...] = jnp.where(lane == idx, -1.0, f)
        return carry

    jax.lax.fori_loop(0, MAXN, out_body, 0)


@jax.jit
def kernel(boxes, scores):
    scores_pad = jnp.pad(scores, ((0, NPAD - N), (0, 0)))
    scoresT = scores_pad.T.reshape(C, R, L)
    boxes_pad = jnp.pad(boxes, ((0, NPAD - N), (0, 0)))
    obox, osc, olab = pl.pallas_call(
        _kern,
        out_shape=[
            jax.ShapeDtypeStruct((MAXN, BD), jnp.float32),
            jax.ShapeDtypeStruct((MAXN, 1), jnp.float32),
            jax.ShapeDtypeStruct((MAXN, 1), jnp.int32),
        ],
        scratch_shapes=[
            pltpu.VMEM((R, L), jnp.float32),
            pltpu.VMEM((R, L), jnp.int32),
            pltpu.VMEM((PRE, BD), jnp.float32),
            pltpu.VMEM((R, 1), jnp.float32),
            pltpu.VMEM((1, PRE), jnp.float32),
            pltpu.VMEM((1, PRE), jnp.float32),
            pltpu.VMEM((1, PRE), jnp.float32),
            pltpu.VMEM((1, PRE), jnp.float32),
            pltpu.VMEM((1, PRE), jnp.float32),
            pltpu.VMEM((1, PRE), jnp.float32),
            pltpu.VMEM((1, PRE), jnp.int32),
            pltpu.VMEM((1, PRE), jnp.float32),
        ],
    )(scoresT, boxes_pad)
    return obox, osc[:, 0], olab[:, 0]


# speedup vs baseline: 1.2027x; 1.2027x over previous
import jax
import jax.numpy as jnp
from jax.experimental import pallas as pl
from jax.experimental.pallas import tpu as pltpu

N = 20000
C = 10
BD = 9
PRE = 1024
THR = 0.5
MAXN = 500
STH = 0.05
NPAD = 20480
R = 160
L = 128


def _kern(scores_ref, boxes_ref,
          obox_ref, oscore_ref, olab_ref,
          m_ref, lab_ref, selbox_ref,
          x1T, x2T, y1T, y2T, areaT, scT, labT, keepT):
    s0 = jax.nn.sigmoid(scores_ref[0])
    m = jnp.where(s0 > STH, s0, 0.0)
    lab = jnp.zeros((R, L), jnp.int32)
    for c in range(1, C):
        sc = jax.nn.sigmoid(scores_ref[c])
        th = jnp.where(sc > STH, sc, 0.0)
        lab = jnp.where(th > m, c, lab)
        m = jnp.maximum(m, th)
    flat = (jax.lax.broadcasted_iota(jnp.int32, (R, L), 0) * L
            + jax.lax.broadcasted_iota(jnp.int32, (R, L), 1))
    m = jnp.where(flat < N, m, -1.0)
    m_ref[...] = m
    lab_ref[...] = lab
    lane128 = jax.lax.broadcasted_iota(jnp.int32, (1, L), 1)

    lane = jax.lax.broadcasted_iota(jnp.int32, (1, PRE), 1)
    zeros = jnp.zeros((1, PRE), jnp.float32)
    x1T[...] = zeros
    x2T[...] = zeros
    y1T[...] = zeros
    y2T[...] = zeros
    areaT[...] = zeros
    scT[...] = zeros
    labT[...] = jnp.zeros((1, PRE), jnp.int32)
    keepT[...] = zeros + 1.0

    def sel_body(k, carry):
        mm = m_ref[...]
        v = jnp.max(mm)
        idx = jnp.min(jnp.where(mm == v, flat, NPAD))
        r = idx // L
        cc = idx - r * L
        row = boxes_ref[pl.ds(idx, 1), :]
        selbox_ref[pl.ds(k, 1), :] = row
        x = row[:, 0:1]
        y = row[:, 1:2]
        w = jnp.abs(row[:, 3:4]) + 1e-3
        ln = jnp.abs(row[:, 4:5]) + 1e-3
        msk = lane == k
        x1T[...] = jnp.where(msk, x - w / 2.0, x1T[...])
        x2T[...] = jnp.where(msk, x + w / 2.0, x2T[...])
        y1T[...] = jnp.where(msk, y - ln / 2.0, y1T[...])
        y2T[...] = jnp.where(msk, y + ln / 2.0, y2T[...])
        areaT[...] = jnp.where(msk, w * ln, areaT[...])
        scT[...] = jnp.where(msk, v, scT[...])
        lab_row = lab_ref[pl.ds(r, 1), :]
        lv = jnp.sum(jnp.where(lane128 == cc, lab_row, 0))
        labT[...] = jnp.where(msk, lv, labT[...])
        srow = m_ref[pl.ds(r, 1), :]
        m_ref[pl.ds(r, 1), :] = jnp.where(lane128 == cc, -1.0, srow)
        return carry

    jax.lax.fori_loop(0, PRE, sel_body, 0)

    def nms_body(i, carry):
        kt = keepT[...]
        onehot = lane == i
        ki = jnp.sum(jnp.where(onehot, kt, 0.0))
        x1i = jnp.sum(jnp.where(onehot, x1T[...], 0.0))
        x2i = jnp.sum(jnp.where(onehot, x2T[...], 0.0))
        y1i = jnp.sum(jnp.where(onehot, y1T[...], 0.0))
        y2i = jnp.sum(jnp.where(onehot, y2T[...], 0.0))
        ai = jnp.sum(jnp.where(onehot, areaT[...], 0.0))
        ix = jnp.maximum(0.0, jnp.minimum(x2i, x2T[...]) - jnp.maximum(x1i, x1T[...]))
        iy = jnp.maximum(0.0, jnp.minimum(y2i, y2T[...]) - jnp.maximum(y1i, y1T[...]))
        inter = ix * iy
        union = ai + areaT[...] - inter
        iou = inter / jnp.maximum(union, 1e-6)
        supp = (iou > THR) & (lane > i) & (ki > 0.0)
        keepT[...] = jnp.where(supp, 0.0, kt)
        return carry

    jax.lax.fori_loop(0, PRE, nms_body, 0)

    scT[...] = jnp.where(keepT[...] > 0.0, scT[...], 0.0)

    def out_body(k, carry):
        f = scT[...]
        v = jnp.max(f)
        idx = jnp.min(jnp.where(f == v, lane, PRE))
        obox_ref[pl.ds(k, 1), :] = selbox_ref[pl.ds(idx, 1), :]
        oscore_ref[pl.ds(k, 1), :] = jnp.full((1, 1), v, jnp.float32)
        lv = jnp.sum(jnp.where(lane == idx, labT[...], 0))
        olab_ref[pl.ds(k, 1), :] = jnp.full((1, 1), lv, jnp.int32)
        scT[...] = jnp.where(lane == idx, -1.0, f)
        return carry

    jax.lax.fori_loop(0, MAXN, out_body, 0)


@jax.jit
def kernel(boxes, scores):
    scores_pad = jnp.pad(scores, ((0, NPAD - N), (0, 0)))
    scoresT = scores_pad.T.reshape(C, R, L)
    boxes_pad = jnp.pad(boxes, ((0, NPAD - N), (0, 0)))
    obox, osc, olab = pl.pallas_call(
        _kern,
        out_shape=[
            jax.ShapeDtypeStruct((MAXN, BD), jnp.float32),
            jax.ShapeDtypeStruct((MAXN, 1), jnp.float32),
            jax.ShapeDtypeStruct((MAXN, 1), jnp.int32),
        ],
        scratch_shapes=[
            pltpu.VMEM((R, L), jnp.float32),
            pltpu.VMEM((R, L), jnp.int32),
            pltpu.VMEM((PRE, BD), jnp.float32),
            pltpu.VMEM((1, PRE), jnp.float32),
            pltpu.VMEM((1, PRE), jnp.float32),
            pltpu.VMEM((1, PRE), jnp.float32),
            pltpu.VMEM((1, PRE), jnp.float32),
            pltpu.VMEM((1, PRE), jnp.float32),
            pltpu.VMEM((1, PRE), jnp.float32),
            pltpu.VMEM((1, PRE), jnp.int32),
            pltpu.VMEM((1, PRE), jnp.float32),
        ],
    )(scoresT, boxes_pad)
    return obox, osc[:, 0], olab[:, 0]
